# baseline (speedup 1.0000x reference)
import functools

import jax
import jax.numpy as jnp
from jax.experimental import pallas as pl
from jax.experimental.pallas import tpu as pltpu

TEMP = 0.05
NUM_CAMS = 8
CHUNK = 1024
NEG = -1e30


def _cam_ce_kernel(x_ref, cams_ref, tgt_ref, p_ref, cid_ref, out_ref,
                   m_ref, s_ref, t_ref, c_ref, *, num_chunks, chunk, b):
    k = pl.program_id(0)

    @pl.when(k == 0)
    def _init():
        m_ref[...] = jnp.full((b, 1), NEG, dtype=jnp.float32)
        s_ref[...] = jnp.zeros((b, 1), dtype=jnp.float32)
        t_ref[...] = jnp.zeros((b, 1), dtype=jnp.float32)
        c_ref[...] = jnp.zeros((NUM_CAMS, 1), dtype=jnp.float32)

    x = x_ref[...]
    norm = jnp.sqrt(jnp.sum(x * x, axis=1, keepdims=True))
    xn = x / (jnp.maximum(norm, 1e-12) * TEMP)
    p = p_ref[...]
    sims = jax.lax.dot_general(
        xn, p, (((1,), (1,)), ((), ())),
        preferred_element_type=jnp.float32)

    cid = cid_ref[0]
    cams = cams_ref[...]
    mask = cams == cid

    cam_iota = jax.lax.broadcasted_iota(jnp.int32, (NUM_CAMS, chunk), 0)
    eq = (cid == cam_iota).astype(jnp.float32)
    jj = jax.lax.broadcasted_iota(jnp.int32, (chunk, chunk), 0)
    kk = jax.lax.broadcasted_iota(jnp.int32, (chunk, chunk), 1)
    lt = (jj <= kk).astype(jnp.float32)
    inc = jax.lax.dot_general(
        eq, lt, (((1,), (0,)), ((), ())),
        preferred_element_type=jnp.float32)
    base = c_ref[...]
    rank = jnp.sum(eq * (inc + base), axis=0, keepdims=True) - 1.0
    c_ref[...] = base + jnp.sum(eq, axis=1, keepdims=True)

    msk = jnp.where(mask, sims, NEG)
    m_old = m_ref[...]
    m_new = jnp.maximum(m_old, jnp.max(msk, axis=1, keepdims=True))
    contrib = jnp.where(mask, jnp.exp(msk - m_new), 0.0)
    s_ref[...] = s_ref[...] * jnp.exp(m_old - m_new) + jnp.sum(
        contrib, axis=1, keepdims=True)
    m_ref[...] = m_new

    tf = tgt_ref[...].astype(jnp.float32)
    tsel = mask & (rank == tf)
    t_ref[...] = t_ref[...] + jnp.sum(jnp.where(tsel, sims, 0.0),
                                      axis=1, keepdims=True)

    @pl.when(k == num_chunks - 1)
    def _fin():
        s = s_ref[...]
        per = jnp.where(s > 0.0,
                        m_ref[...] + jnp.log(s) - t_ref[...], 0.0)
        out_ref[...] = jnp.sum(per, axis=0, keepdims=True) / b


def kernel(inputs, targets, cams, proxy, pids, cids):
    del pids
    b, d = inputs.shape
    s = proxy.shape[0]
    num_chunks = -(-s // CHUNK)
    spad = num_chunks * CHUNK
    proxy_p = jnp.pad(proxy, ((0, spad - s), (0, 0)))
    cids_p = jnp.pad(cids.astype(jnp.int32), (0, spad - s),
                     constant_values=NUM_CAMS)
    cids3 = cids_p.reshape(num_chunks, 1, CHUNK)
    cams2 = cams.astype(jnp.int32).reshape(b, 1)
    tgts2 = targets.astype(jnp.int32).reshape(b, 1)

    grid = (num_chunks,)
    out = pl.pallas_call(
        functools.partial(_cam_ce_kernel, num_chunks=num_chunks,
                          chunk=CHUNK, b=b),
        grid=grid,
        in_specs=[
            pl.BlockSpec((b, d), lambda k: (0, 0)),
            pl.BlockSpec((b, 1), lambda k: (0, 0)),
            pl.BlockSpec((b, 1), lambda k: (0, 0)),
            pl.BlockSpec((CHUNK, d), lambda k: (k, 0)),
            pl.BlockSpec((1, 1, CHUNK), lambda k: (k, 0, 0)),
        ],
        out_specs=pl.BlockSpec((1, 1), lambda k: (0, 0)),
        out_shape=jax.ShapeDtypeStruct((1, 1), jnp.float32),
        scratch_shapes=[
            pltpu.VMEM((b, 1), jnp.float32),
            pltpu.VMEM((b, 1), jnp.float32),
            pltpu.VMEM((b, 1), jnp.float32),
            pltpu.VMEM((NUM_CAMS, 1), jnp.float32),
        ],
    )(inputs, cams2, tgts2, proxy_p, cids3)
    return out.reshape(1)


# speedup vs baseline: 1.1009x; 1.1009x over previous
import functools

import jax
import jax.numpy as jnp
from jax.experimental import pallas as pl
from jax.experimental.pallas import tpu as pltpu

TEMP = 0.05
NUM_CAMS = 8
CHUNK = 2048
NEG = -1e30


def _cam_ce_kernel(x_ref, cams_ref, tgt_ref, p_ref, cid_ref, out_ref,
                   m_ref, s_ref, t_ref, c_ref, xn_ref, lt_ref,
                   *, num_chunks, chunk, b):
    k = pl.program_id(0)

    @pl.when(k == 0)
    def _init():
        m_ref[...] = jnp.full((b, 1), NEG, dtype=jnp.float32)
        s_ref[...] = jnp.zeros((b, 1), dtype=jnp.float32)
        t_ref[...] = jnp.zeros((b, 1), dtype=jnp.float32)
        c_ref[...] = jnp.zeros((NUM_CAMS, 1), dtype=jnp.float32)
        x = x_ref[...]
        norm = jnp.sqrt(jnp.sum(x * x, axis=1, keepdims=True))
        xn_ref[...] = (x / (jnp.maximum(norm, 1e-12) * TEMP)
                       ).astype(jnp.bfloat16)
        jj = jax.lax.broadcasted_iota(jnp.int32, (chunk, chunk), 0)
        kk = jax.lax.broadcasted_iota(jnp.int32, (chunk, chunk), 1)
        lt_ref[...] = (jj <= kk).astype(jnp.bfloat16)

    xn = xn_ref[...]
    p = p_ref[...].astype(jnp.bfloat16)
    sims = jax.lax.dot_general(
        xn, p, (((1,), (1,)), ((), ())),
        preferred_element_type=jnp.float32)

    cid = cid_ref[0]
    cams = cams_ref[...]
    mask = cams == cid

    cam_iota = jax.lax.broadcasted_iota(jnp.int32, (NUM_CAMS, chunk), 0)
    eq = (cid == cam_iota).astype(jnp.bfloat16)
    inc = jax.lax.dot_general(
        eq, lt_ref[...], (((1,), (0,)), ((), ())),
        preferred_element_type=jnp.float32)
    eq = eq.astype(jnp.float32)
    base = c_ref[...]
    rank = jnp.sum(eq * (inc + base), axis=0, keepdims=True) - 1.0
    c_ref[...] = base + jnp.sum(eq, axis=1, keepdims=True)

    msk = jnp.where(mask, sims, NEG)
    m_old = m_ref[...]
    m_new = jnp.maximum(m_old, jnp.max(msk, axis=1, keepdims=True))
    contrib = jnp.where(mask, jnp.exp(msk - m_new), 0.0)
    s_ref[...] = s_ref[...] * jnp.exp(m_old - m_new) + jnp.sum(
        contrib, axis=1, keepdims=True)
    m_ref[...] = m_new

    tf = tgt_ref[...].astype(jnp.float32)
    tsel = mask & (rank == tf)
    t_ref[...] = t_ref[...] + jnp.sum(jnp.where(tsel, sims, 0.0),
                                      axis=1, keepdims=True)

    @pl.when(k == num_chunks - 1)
    def _fin():
        s = s_ref[...]
        per = jnp.where(s > 0.0,
                        m_ref[...] + jnp.log(s) - t_ref[...], 0.0)
        out_ref[...] = jnp.sum(per, axis=0, keepdims=True) / b


def kernel(inputs, targets, cams, proxy, pids, cids):
    del pids
    b, d = inputs.shape
    s = proxy.shape[0]
    num_chunks = -(-s // CHUNK)
    spad = num_chunks * CHUNK
    proxy_p = jnp.pad(proxy, ((0, spad - s), (0, 0)))
    cids_p = jnp.pad(cids.astype(jnp.int32), (0, spad - s),
                     constant_values=NUM_CAMS)
    cids3 = cids_p.reshape(num_chunks, 1, CHUNK)
    cams2 = cams.astype(jnp.int32).reshape(b, 1)
    tgts2 = targets.astype(jnp.int32).reshape(b, 1)

    grid = (num_chunks,)
    out = pl.pallas_call(
        functools.partial(_cam_ce_kernel, num_chunks=num_chunks,
                          chunk=CHUNK, b=b),
        grid=grid,
        in_specs=[
            pl.BlockSpec((b, d), lambda k: (0, 0)),
            pl.BlockSpec((b, 1), lambda k: (0, 0)),
            pl.BlockSpec((b, 1), lambda k: (0, 0)),
            pl.BlockSpec((CHUNK, d), lambda k: (k, 0)),
            pl.BlockSpec((1, 1, CHUNK), lambda k: (k, 0, 0)),
        ],
        out_specs=pl.BlockSpec((1, 1), lambda k: (0, 0)),
        out_shape=jax.ShapeDtypeStruct((1, 1), jnp.float32),
        scratch_shapes=[
            pltpu.VMEM((b, 1), jnp.float32),
            pltpu.VMEM((b, 1), jnp.float32),
            pltpu.VMEM((b, 1), jnp.float32),
            pltpu.VMEM((NUM_CAMS, 1), jnp.float32),
            pltpu.VMEM((b, d), jnp.bfloat16),
            pltpu.VMEM((CHUNK, CHUNK), jnp.bfloat16),
        ],
    )(inputs, cams2, tgts2, proxy_p, cids3)
    return out.reshape(1)


# speedup vs baseline: 1.4617x; 1.3277x over previous
import functools

import jax
import jax.numpy as jnp
from jax.experimental import pallas as pl
from jax.experimental.pallas import tpu as pltpu

TEMP = 0.05
NUM_CAMS = 8
CHUNK = 2048
NEG = -1e30


def _cam_ce_kernel(x_ref, cams_ref, tgt_ref, p_ref, cid_ref, out_ref,
                   m_ref, s_ref, t_ref, c_ref, xn_ref, lt_ref, oh_ref,
                   *, num_chunks, chunk, b):
    k = pl.program_id(0)

    @pl.when(k == 0)
    def _init():
        m_ref[...] = jnp.full((b, 1), NEG, dtype=jnp.float32)
        s_ref[...] = jnp.zeros((b, 1), dtype=jnp.float32)
        t_ref[...] = jnp.zeros((b, 1), dtype=jnp.float32)
        c_ref[...] = jnp.zeros((NUM_CAMS, 1), dtype=jnp.float32)
        x = x_ref[...]
        norm = jnp.sqrt(jnp.sum(x * x, axis=1, keepdims=True))
        xn_ref[...] = (x / (jnp.maximum(norm, 1e-12) * TEMP)
                       ).astype(jnp.bfloat16)
        jj = jax.lax.broadcasted_iota(jnp.int32, (chunk, chunk), 0)
        kk = jax.lax.broadcasted_iota(jnp.int32, (chunk, chunk), 1)
        lt_ref[...] = (jj <= kk).astype(jnp.bfloat16)
        ci = jax.lax.broadcasted_iota(jnp.int32, (b, NUM_CAMS), 1)
        oh_ref[...] = (cams_ref[...] == ci).astype(jnp.float32)

    p = p_ref[...].astype(jnp.bfloat16)
    sims = jax.lax.dot_general(
        xn_ref[...], p, (((1,), (1,)), ((), ())),
        preferred_element_type=jnp.float32)

    cid = cid_ref[0]
    cam_iota = jax.lax.broadcasted_iota(jnp.int32, (NUM_CAMS, chunk), 0)
    eqh = (cid == cam_iota).astype(jnp.bfloat16)
    inc = jax.lax.dot_general(
        eqh, lt_ref[...], (((1,), (0,)), ((), ())),
        preferred_element_type=jnp.float32)
    eq = eqh.astype(jnp.float32)
    base = c_ref[...]
    rank = jnp.sum(eq * (inc + base), axis=0, keepdims=True) - 1.0
    c_ref[...] = base + jnp.sum(eq, axis=1, keepdims=True)

    m_old = m_ref[...]
    m_new = jnp.maximum(m_old, jnp.max(sims, axis=1, keepdims=True))
    e = jnp.exp(sims - m_new)
    e8 = jax.lax.dot_general(
        e, eq, (((1,), (1,)), ((), ())),
        preferred_element_type=jnp.float32)
    oh = oh_ref[...]
    s_ref[...] = (s_ref[...] * jnp.exp(m_old - m_new)
                  + jnp.sum(e8 * oh, axis=1, keepdims=True))
    m_ref[...] = m_new

    tf = tgt_ref[...].astype(jnp.float32)
    tmp = jnp.where(rank == tf, sims, 0.0)
    t8 = jax.lax.dot_general(
        tmp, eq, (((1,), (1,)), ((), ())),
        preferred_element_type=jnp.float32)
    t_ref[...] = t_ref[...] + jnp.sum(t8 * oh, axis=1, keepdims=True)

    @pl.when(k == num_chunks - 1)
    def _fin():
        s = s_ref[...]
        per = jnp.where(s > 0.0,
                        m_ref[...] + jnp.log(s) - t_ref[...], 0.0)
        out_ref[...] = jnp.sum(per, axis=0, keepdims=True) / b


def kernel(inputs, targets, cams, proxy, pids, cids):
    del pids
    b, d = inputs.shape
    s = proxy.shape[0]
    num_chunks = -(-s // CHUNK)
    spad = num_chunks * CHUNK
    proxy_p = jnp.pad(proxy, ((0, spad - s), (0, 0)))
    cids_p = jnp.pad(cids.astype(jnp.int32), (0, spad - s),
                     constant_values=NUM_CAMS)
    cids3 = cids_p.reshape(num_chunks, 1, CHUNK)
    cams2 = cams.astype(jnp.int32).reshape(b, 1)
    tgts2 = targets.astype(jnp.int32).reshape(b, 1)

    grid = (num_chunks,)
    out = pl.pallas_call(
        functools.partial(_cam_ce_kernel, num_chunks=num_chunks,
                          chunk=CHUNK, b=b),
        grid=grid,
        in_specs=[
            pl.BlockSpec((b, d), lambda k: (0, 0)),
            pl.BlockSpec((b, 1), lambda k: (0, 0)),
            pl.BlockSpec((b, 1), lambda k: (0, 0)),
            pl.BlockSpec((CHUNK, d), lambda k: (k, 0)),
            pl.BlockSpec((1, 1, CHUNK), lambda k: (k, 0, 0)),
        ],
        out_specs=pl.BlockSpec((1, 1), lambda k: (0, 0)),
        out_shape=jax.ShapeDtypeStruct((1, 1), jnp.float32),
        scratch_shapes=[
            pltpu.VMEM((b, 1), jnp.float32),
            pltpu.VMEM((b, 1), jnp.float32),
            pltpu.VMEM((b, 1), jnp.float32),
            pltpu.VMEM((NUM_CAMS, 1), jnp.float32),
            pltpu.VMEM((b, d), jnp.bfloat16),
            pltpu.VMEM((CHUNK, CHUNK), jnp.bfloat16),
            pltpu.VMEM((b, NUM_CAMS), jnp.float32),
        ],
    )(inputs, cams2, tgts2, proxy_p, cids3)
    return out.reshape(1)
